# Initial kernel scaffold; baseline (speedup 1.0000x reference)
import functools

import jax
import jax.numpy as jnp
from jax import lax
from jax.experimental import pallas as pl
from jax.experimental.pallas import tpu as pltpu
from jax.experimental.pallas import tpu_sc as plsc

N = 10000
E = 320000
D_IN = 128
H = 8
F = 16
HF = H * F

NC = 2
NS = 16
NW = NC * NS
B = 128
NB = -(-E // (NW * B))
E_PAD = NW * NB * B
N_PAD = 10240
_RPS = N_PAD // NS


def _pre_body(x_ref, w_ref, wskip_ref, asrc_ref, atrg_ref,
              proj_ref, ssrc_ref, strg_ref, skip_ref):
    xb = x_ref[...]
    p = jnp.dot(xb, w_ref[...], preferred_element_type=jnp.float32)
    proj_ref[...] = p
    ssrc_ref[...] = jnp.dot(p, asrc_ref[...], preferred_element_type=jnp.float32)
    strg_ref[...] = jnp.dot(p, atrg_ref[...], preferred_element_type=jnp.float32)
    skip_ref[...] = jnp.dot(xb, wskip_ref[...], preferred_element_type=jnp.float32)


def _post_body(n0_ref, n1_ref, d0_ref, d1_ref, skip_ref, bias_ref, exp_ref,
               out_ref):
    den = d0_ref[...] + d1_ref[...]
    dinv = 1.0 / (den + 1e-16)
    dinv128 = jnp.dot(dinv, exp_ref[...], preferred_element_type=jnp.float32)
    v = (n0_ref[...] + n1_ref[...]) * dinv128 + skip_ref[...] + bias_ref[...]
    out_ref[...] = jnp.where(v > 0.0, v, jnp.expm1(jnp.minimum(v, 0.0)))


def _sc_body(src_hbm, trgg_hbm, trgs_hbm, ssrc_hbm, strg_hbm, proj_hbm,
             num_hbm, den_hbm,
             sh_num, sh_den, zbuf, zden, prow, ssrc, strg, exps,
             srcv, trgg, trgs, sem):
    c = lax.axis_index("c")
    s = lax.axis_index("s")

    zeros = jnp.zeros((16,), jnp.float32)

    @pl.loop(0, 320)
    def _zero_rows(r):
        for j in range(8):
            zbuf[r, pl.ds(j * 16, 16)] = zeros
        zden[r, :] = zeros

    r0 = s * 320
    pltpu.sync_copy(zbuf, sh_num.at[pl.ds(r0, 320)])
    pltpu.sync_copy(zden, sh_den.at[pl.ds(r0, 320)])
    r1 = (NS + s) * 320
    pltpu.sync_copy(zbuf, sh_num.at[pl.ds(r1, 320)])
    pltpu.sync_copy(zden, sh_den.at[pl.ds(r1, 320)])

    plsc.subcore_barrier()

    base = (c * NS + s) * (NB * B)

    @pl.loop(0, NB)
    def _edge_block(blk):
        off = base + blk * B
        pltpu.sync_copy(src_hbm.at[pl.ds(off, B)], srcv)
        pltpu.sync_copy(trgg_hbm.at[pl.ds(off, B)], trgg)
        pltpu.sync_copy(trgs_hbm.at[pl.ds(off, B)], trgs)
        cp1 = pltpu.async_copy(ssrc_hbm.at[srcv], ssrc, sem)
        cp2 = pltpu.async_copy(strg_hbm.at[trgg], strg, sem)
        cp3 = pltpu.async_copy(proj_hbm.at[srcv], prow, sem)
        cp1.wait()
        cp2.wait()
        cp3.wait()

        @pl.loop(0, B)
        def _edge(e):
            a = ssrc[e, :]
            b = strg[e, :]
            z = a + b
            zl = jnp.where(z > 0.0, z, 0.2 * z)
            exps[e, :] = jnp.exp(zl)
            for h in range(H):
                sc = exps[e, h]
                prow[e, pl.ds(h * F, F)] = prow[e, pl.ds(h * F, F)] * sc

        pltpu.sync_copy(exps, sh_den.at[trgs], add=True)
        pltpu.sync_copy(prow, sh_num.at[trgs], add=True)

    plsc.subcore_barrier()

    w0 = s * _RPS
    pltpu.sync_copy(sh_num.at[pl.ds(w0, _RPS)], num_hbm.at[c, pl.ds(w0, _RPS), :])
    pltpu.sync_copy(sh_den.at[pl.ds(w0, _RPS)], den_hbm.at[c, pl.ds(w0, _RPS), :])


_sc_edge = pl.kernel(
    _sc_body,
    out_type=[
        jax.ShapeDtypeStruct((NC, N_PAD, HF), jnp.float32),
        jax.ShapeDtypeStruct((NC, N_PAD, 16), jnp.float32),
    ],
    mesh=plsc.VectorSubcoreMesh(core_axis_name="c", subcore_axis_name="s"),
    scratch_types=[
        pltpu.VMEM_SHARED((N_PAD, HF), jnp.float32),
        pltpu.VMEM_SHARED((N_PAD, 16), jnp.float32),
        pltpu.VMEM((320, HF), jnp.float32),
        pltpu.VMEM((320, 16), jnp.float32),
        pltpu.VMEM((B, HF), jnp.float32),
        pltpu.VMEM((B, 16), jnp.float32),
        pltpu.VMEM((B, 16), jnp.float32),
        pltpu.VMEM((B, 16), jnp.float32),
        pltpu.VMEM((B,), jnp.int32),
        pltpu.VMEM((B,), jnp.int32),
        pltpu.VMEM((B,), jnp.int32),
        pltpu.SemaphoreType.DMA,
    ],
)


def kernel(x, edge_index, W, a_src, a_trg, W_skip, bias):
    f32 = jnp.float32

    eye_mask = jnp.kron(jnp.eye(H, dtype=f32), jnp.ones((F, 1), f32))
    a_src_m = jnp.pad(eye_mask * a_src.reshape(HF, 1), ((0, 0), (0, 8)))
    a_trg_m = jnp.pad(eye_mask * a_trg.reshape(HF, 1), ((0, 0), (0, 8)))
    expand = jnp.concatenate(
        [jnp.kron(jnp.eye(H, dtype=f32), jnp.ones((1, F), f32)),
         jnp.zeros((8, HF), f32)], axis=0)

    R = 1000
    grid = N // R
    proj, ssrc, strg, skip = pl.pallas_call(
        _pre_body,
        grid=(grid,),
        in_specs=[
            pl.BlockSpec((R, D_IN), lambda i: (i, 0)),
            pl.BlockSpec((D_IN, HF), lambda i: (0, 0)),
            pl.BlockSpec((D_IN, HF), lambda i: (0, 0)),
            pl.BlockSpec((HF, 16), lambda i: (0, 0)),
            pl.BlockSpec((HF, 16), lambda i: (0, 0)),
        ],
        out_specs=[
            pl.BlockSpec((R, HF), lambda i: (i, 0)),
            pl.BlockSpec((R, 16), lambda i: (i, 0)),
            pl.BlockSpec((R, 16), lambda i: (i, 0)),
            pl.BlockSpec((R, HF), lambda i: (i, 0)),
        ],
        out_shape=[
            jax.ShapeDtypeStruct((N, HF), f32),
            jax.ShapeDtypeStruct((N, 16), f32),
            jax.ShapeDtypeStruct((N, 16), f32),
            jax.ShapeDtypeStruct((N, HF), f32),
        ],
    )(x, W, W_skip, a_src_m, a_trg_m)

    src = edge_index[0]
    trg = edge_index[1]
    pad = E_PAD - E
    src_p = jnp.concatenate([src, jnp.zeros((pad,), jnp.int32)])
    trg_g = jnp.concatenate([trg, jnp.zeros((pad,), jnp.int32)])
    trg_s = jnp.concatenate([trg, jnp.full((pad,), N, jnp.int32)])

    num, den = _sc_edge(src_p, trg_g, trg_s, ssrc, strg, proj)

    out = pl.pallas_call(
        _post_body,
        grid=(grid,),
        in_specs=[
            pl.BlockSpec((R, HF), lambda i: (i, 0)),
            pl.BlockSpec((R, HF), lambda i: (i, 0)),
            pl.BlockSpec((R, 16), lambda i: (i, 0)),
            pl.BlockSpec((R, 16), lambda i: (i, 0)),
            pl.BlockSpec((R, HF), lambda i: (i, 0)),
            pl.BlockSpec((1, HF), lambda i: (0, 0)),
            pl.BlockSpec((16, HF), lambda i: (0, 0)),
        ],
        out_specs=pl.BlockSpec((R, HF), lambda i: (i, 0)),
        out_shape=jax.ShapeDtypeStruct((N, HF), f32),
    )(num[0, :N], num[1, :N], den[0, :N], den[1, :N], skip,
      bias.reshape(1, HF), expand)
    return out


# speedup vs baseline: 51.1823x; 51.1823x over previous
import functools

import jax
import jax.numpy as jnp
from jax import lax
from jax.experimental import pallas as pl
from jax.experimental.pallas import tpu as pltpu
from jax.experimental.pallas import tpu_sc as plsc

N = 10000
E = 320000
D_IN = 128
H = 8
F = 16
HF = H * F

NC = 2
NS = 16
NW = NC * NS
B = 128
NB = -(-E // (NW * B))
E_PAD = NW * NB * B
N_PAD = 10240
_RPS = N_PAD // NS


def _pre_body(x_ref, w_ref, wskip_ref, ab_ref,
              proj_ref, sboth_ref, skip_ref):
    xb = x_ref[...]
    p = jnp.dot(xb, w_ref[...], preferred_element_type=jnp.float32)
    proj_ref[...] = p
    sboth_ref[...] = jnp.dot(p, ab_ref[...], preferred_element_type=jnp.float32)
    skip_ref[...] = jnp.dot(xb, wskip_ref[...], preferred_element_type=jnp.float32)


def _post_body(n0_ref, n1_ref, d0_ref, d1_ref, skip_ref, bias_ref, exp_ref,
               out_ref):
    den = d0_ref[...] + d1_ref[...]
    dinv = 1.0 / (den + 1e-16)
    dinv128 = jnp.dot(dinv, exp_ref[...], preferred_element_type=jnp.float32)
    v = (n0_ref[...] + n1_ref[...]) * dinv128 + skip_ref[...] + bias_ref[...]
    out_ref[...] = jnp.where(v > 0.0, v, jnp.exp(jnp.minimum(v, 0.0)) - 1.0)


def _sc_body(src_hbm, trgg_hbm, trgs_hbm, sboth_hbm, proj_hbm,
             num_hbm, den_hbm,
             sh_num, sh_den, zbuf, zden, prow, ssrc, strg, exps,
             srcv, trgg, trgs, sem):
    c = lax.axis_index("c")
    s = lax.axis_index("s")

    zeros = jnp.zeros((16,), jnp.float32)

    @pl.loop(0, 16)
    def _zero_rows(r):
        for j in range(8):
            zbuf[r, pl.ds(j * 16, 16)] = zeros
        zden[r, :] = zeros

    r0 = s * 320
    r1 = (NS + s) * 320

    @pl.loop(0, 20)
    def _zero_copy(t):
        pltpu.sync_copy(zbuf, sh_num.at[pl.ds(r0 + t * 16, 16)])
        pltpu.sync_copy(zden, sh_den.at[pl.ds(r0 + t * 16, 16)])
        pltpu.sync_copy(zbuf, sh_num.at[pl.ds(r1 + t * 16, 16)])
        pltpu.sync_copy(zden, sh_den.at[pl.ds(r1 + t * 16, 16)])

    plsc.subcore_barrier()

    base = (c * NS + s) * (NB * B)

    @pl.loop(0, NB)
    def _edge_block(blk):
        off = base + blk * B
        pltpu.sync_copy(src_hbm.at[pl.ds(off, B)], srcv)
        pltpu.sync_copy(trgg_hbm.at[pl.ds(off, B)], trgg)
        pltpu.sync_copy(trgs_hbm.at[pl.ds(off, B)], trgs)
        cp1 = pltpu.async_copy(sboth_hbm.at[srcv], ssrc, sem)
        cp2 = pltpu.async_copy(sboth_hbm.at[trgg], strg, sem)
        cp3 = pltpu.async_copy(proj_hbm.at[srcv], prow, sem)
        cp1.wait()
        cp2.wait()
        cp3.wait()

        perm = jnp.minimum(lax.iota(jnp.int32, 16) + 8, 15)
        dnums = lax.GatherDimensionNumbers(
            offset_dims=(), collapsed_slice_dims=(0,), start_index_map=(0,))

        @pl.loop(0, B)
        def _edge(e):
            a = ssrc[e, :]
            b = strg[e, :]
            bp = lax.gather(b, perm[:, None], dnums, (1,),
                            mode=lax.GatherScatterMode.PROMISE_IN_BOUNDS)
            z = a + bp
            zl = jnp.where(z > 0.0, z, 0.2 * z)
            ex = jnp.exp(zl)
            exps[e, :] = ex
            for h in range(H):
                sc = ex[h]
                prow[e, pl.ds(h * F, F)] = prow[e, pl.ds(h * F, F)] * sc

        pltpu.sync_copy(exps, sh_den.at[trgs], add=True)
        pltpu.sync_copy(prow, sh_num.at[trgs], add=True)

    plsc.subcore_barrier()

    w0 = s * _RPS
    pltpu.sync_copy(sh_num.at[pl.ds(w0, _RPS)], num_hbm.at[c, pl.ds(w0, _RPS), :])
    pltpu.sync_copy(sh_den.at[pl.ds(w0, _RPS)], den_hbm.at[c, pl.ds(w0, _RPS), :])


_sc_edge = pl.kernel(
    _sc_body,
    out_type=[
        jax.ShapeDtypeStruct((NC, N_PAD, HF), jnp.float32),
        jax.ShapeDtypeStruct((NC, N_PAD, 16), jnp.float32),
    ],
    mesh=plsc.VectorSubcoreMesh(core_axis_name="c", subcore_axis_name="s"),
    compiler_params=pltpu.CompilerParams(use_tc_tiling_on_sc=False),
    scratch_types=[
        pltpu.VMEM_SHARED((N_PAD, HF), jnp.float32),
        pltpu.VMEM_SHARED((N_PAD, 16), jnp.float32),
        pltpu.VMEM((16, HF), jnp.float32),
        pltpu.VMEM((16, 16), jnp.float32),
        pltpu.VMEM((B, HF), jnp.float32),
        pltpu.VMEM((B, 16), jnp.float32),
        pltpu.VMEM((B, 16), jnp.float32),
        pltpu.VMEM((B, 16), jnp.float32),
        pltpu.VMEM((B,), jnp.int32),
        pltpu.VMEM((B,), jnp.int32),
        pltpu.VMEM((B,), jnp.int32),
        pltpu.SemaphoreType.DMA,
    ],
)


def kernel(x, edge_index, W, a_src, a_trg, W_skip, bias):
    f32 = jnp.float32

    eye_mask = jnp.kron(jnp.eye(H, dtype=f32), jnp.ones((F, 1), f32))
    a_src_m = eye_mask * a_src.reshape(HF, 1)
    a_trg_m = eye_mask * a_trg.reshape(HF, 1)
    ab = jnp.concatenate([a_src_m, a_trg_m], axis=1)
    expand = jnp.concatenate(
        [jnp.kron(jnp.eye(H, dtype=f32), jnp.ones((1, F), f32)),
         jnp.zeros((8, HF), f32)], axis=0)

    R = 1000
    grid = N // R
    proj, sboth, skip = pl.pallas_call(
        _pre_body,
        grid=(grid,),
        in_specs=[
            pl.BlockSpec((R, D_IN), lambda i: (i, 0)),
            pl.BlockSpec((D_IN, HF), lambda i: (0, 0)),
            pl.BlockSpec((D_IN, HF), lambda i: (0, 0)),
            pl.BlockSpec((HF, 16), lambda i: (0, 0)),
        ],
        out_specs=[
            pl.BlockSpec((R, HF), lambda i: (i, 0)),
            pl.BlockSpec((R, 16), lambda i: (i, 0)),
            pl.BlockSpec((R, HF), lambda i: (i, 0)),
        ],
        out_shape=[
            jax.ShapeDtypeStruct((N, HF), f32),
            jax.ShapeDtypeStruct((N, 16), f32),
            jax.ShapeDtypeStruct((N, HF), f32),
        ],
    )(x, W, W_skip, ab)

    src = edge_index[0]
    trg = edge_index[1]
    pad = E_PAD - E
    src_p = jnp.concatenate([src, jnp.zeros((pad,), jnp.int32)])
    trg_g = jnp.concatenate([trg, jnp.zeros((pad,), jnp.int32)])
    trg_s = jnp.concatenate([trg, jnp.full((pad,), N, jnp.int32)])

    num, den = _sc_edge(src_p, trg_g, trg_s, sboth, proj)

    out = pl.pallas_call(
        _post_body,
        grid=(grid,),
        in_specs=[
            pl.BlockSpec((R, HF), lambda i: (i, 0)),
            pl.BlockSpec((R, HF), lambda i: (i, 0)),
            pl.BlockSpec((R, 16), lambda i: (i, 0)),
            pl.BlockSpec((R, 16), lambda i: (i, 0)),
            pl.BlockSpec((R, HF), lambda i: (i, 0)),
            pl.BlockSpec((1, HF), lambda i: (0, 0)),
            pl.BlockSpec((16, HF), lambda i: (0, 0)),
        ],
        out_specs=pl.BlockSpec((R, HF), lambda i: (i, 0)),
        out_shape=jax.ShapeDtypeStruct((N, HF), f32),
    )(num[0, :N], num[1, :N], den[0, :N], den[1, :N], skip,
      bias.reshape(1, HF), expand)
    return out


# speedup vs baseline: 56.4057x; 1.1021x over previous
import functools

import jax
import jax.numpy as jnp
from jax import lax
from jax.experimental import pallas as pl
from jax.experimental.pallas import tpu as pltpu
from jax.experimental.pallas import tpu_sc as plsc

N = 10000
E = 320000
D_IN = 128
H = 8
F = 16
HF = H * F

NC = 2
NS = 16
NW = NC * NS
B = 64
NB = 2 * (-(-E // (NW * B * 2)))
E_PAD = NW * NB * B
E_ALLOC = E_PAD + B
N_PAD = 10240
_RPS = N_PAD // NS


def _pre_body(x_ref, w_ref, wskip_ref, ab_ref,
              proj_ref, sboth_ref, skip_ref):
    xb = x_ref[...]
    p = jnp.dot(xb, w_ref[...], preferred_element_type=jnp.float32)
    proj_ref[...] = p
    sboth_ref[...] = jnp.dot(p, ab_ref[...], preferred_element_type=jnp.float32)
    skip_ref[...] = jnp.dot(xb, wskip_ref[...], preferred_element_type=jnp.float32)


def _post_body(n0_ref, n1_ref, d0_ref, d1_ref, skip_ref, bias_ref, exp_ref,
               out_ref):
    den = d0_ref[...] + d1_ref[...]
    dinv = 1.0 / (den + 1e-16)
    dinv128 = jnp.dot(dinv, exp_ref[...], preferred_element_type=jnp.float32)
    v = (n0_ref[...] + n1_ref[...]) * dinv128 + skip_ref[...] + bias_ref[...]
    out_ref[...] = jnp.where(v > 0.0, v, jnp.exp(jnp.minimum(v, 0.0)) - 1.0)


def _sc_body(src_hbm, trgg_hbm, trgs_hbm, sboth_hbm, proj_hbm,
             num_hbm, den_hbm,
             sh_num, sh_den, zbuf, zden,
             prow0, ssrc0, strg0, srcv0, trgg0, trgs0, sem0,
             prow1, ssrc1, strg1, srcv1, trgg1, trgs1, sem1,
             exps):
    c = lax.axis_index("c")
    s = lax.axis_index("s")

    zeros = jnp.zeros((16,), jnp.float32)

    @pl.loop(0, 16)
    def _zero_rows(r):
        for j in range(8):
            zbuf[r, pl.ds(j * 16, 16)] = zeros
        zden[r, :] = zeros

    r0 = s * 320
    r1 = (NS + s) * 320

    @pl.loop(0, 20)
    def _zero_copy(t):
        pltpu.sync_copy(zbuf, sh_num.at[pl.ds(r0 + t * 16, 16)])
        pltpu.sync_copy(zden, sh_den.at[pl.ds(r0 + t * 16, 16)])
        pltpu.sync_copy(zbuf, sh_num.at[pl.ds(r1 + t * 16, 16)])
        pltpu.sync_copy(zden, sh_den.at[pl.ds(r1 + t * 16, 16)])

    plsc.subcore_barrier()

    base = (c * NS + s) * (NB * B)
    bufs = ((prow0, ssrc0, strg0, srcv0, trgg0, trgs0, sem0),
            (prow1, ssrc1, strg1, srcv1, trgg1, trgs1, sem1))

    def _fetch(blk, bset):
        prow, ssrc, strg, srcv, trgg, trgs, sem = bset
        off = base + blk * B
        pltpu.sync_copy(src_hbm.at[pl.ds(off, B)], srcv)
        pltpu.sync_copy(trgg_hbm.at[pl.ds(off, B)], trgg)
        pltpu.sync_copy(trgs_hbm.at[pl.ds(off, B)], trgs)
        pltpu.async_copy(sboth_hbm.at[srcv], ssrc, sem)
        pltpu.async_copy(sboth_hbm.at[trgg], strg, sem)
        pltpu.async_copy(proj_hbm.at[srcv], prow, sem)

    def _drain(bset):
        prow, ssrc, strg, srcv, trgg, trgs, sem = bset
        pltpu.make_async_copy(sboth_hbm.at[srcv], ssrc, sem).wait()
        pltpu.make_async_copy(sboth_hbm.at[trgg], strg, sem).wait()
        pltpu.make_async_copy(proj_hbm.at[srcv], prow, sem).wait()

    perm = jnp.minimum(lax.iota(jnp.int32, 16) + 8, 15)
    dnums = lax.GatherDimensionNumbers(
        offset_dims=(), collapsed_slice_dims=(0,), start_index_map=(0,))

    def _consume(bset):
        prow, ssrc, strg, srcv, trgg, trgs, sem = bset

        @pl.loop(0, B, unroll=2)
        def _edge(e):
            a = ssrc[e, :]
            b = strg[e, :]
            bp = lax.gather(b, perm[:, None], dnums, (1,),
                            mode=lax.GatherScatterMode.PROMISE_IN_BOUNDS)
            z = a + bp
            zl = jnp.where(z > 0.0, z, 0.2 * z)
            ex = jnp.exp(zl)
            exps[e, :] = ex
            for h in range(H):
                sc = ex[h]
                prow[e, pl.ds(h * F, F)] = prow[e, pl.ds(h * F, F)] * sc

        pltpu.sync_copy(exps, sh_den.at[trgs], add=True)
        pltpu.sync_copy(prow, sh_num.at[trgs], add=True)

    _fetch(0, bufs[0])

    @pl.loop(0, NB, step=2)
    def _edge_block(blk):
        _drain(bufs[0])
        _fetch(blk + 1, bufs[1])
        _consume(bufs[0])
        _drain(bufs[1])
        _fetch(blk + 2, bufs[0])
        _consume(bufs[1])

    _drain(bufs[0])

    plsc.subcore_barrier()

    w0 = s * _RPS
    pltpu.sync_copy(sh_num.at[pl.ds(w0, _RPS)], num_hbm.at[c, pl.ds(w0, _RPS), :])
    pltpu.sync_copy(sh_den.at[pl.ds(w0, _RPS)], den_hbm.at[c, pl.ds(w0, _RPS), :])


_sc_edge = pl.kernel(
    _sc_body,
    out_type=[
        jax.ShapeDtypeStruct((NC, N_PAD, HF), jnp.float32),
        jax.ShapeDtypeStruct((NC, N_PAD, 16), jnp.float32),
    ],
    mesh=plsc.VectorSubcoreMesh(core_axis_name="c", subcore_axis_name="s"),
    compiler_params=pltpu.CompilerParams(use_tc_tiling_on_sc=False),
    scratch_types=[
        pltpu.VMEM_SHARED((N_PAD, HF), jnp.float32),
        pltpu.VMEM_SHARED((N_PAD, 16), jnp.float32),
        pltpu.VMEM((16, HF), jnp.float32),
        pltpu.VMEM((16, 16), jnp.float32),
        pltpu.VMEM((B, HF), jnp.float32),
        pltpu.VMEM((B, 16), jnp.float32),
        pltpu.VMEM((B, 16), jnp.float32),
        pltpu.VMEM((B,), jnp.int32),
        pltpu.VMEM((B,), jnp.int32),
        pltpu.VMEM((B,), jnp.int32),
        pltpu.SemaphoreType.DMA,
        pltpu.VMEM((B, HF), jnp.float32),
        pltpu.VMEM((B, 16), jnp.float32),
        pltpu.VMEM((B, 16), jnp.float32),
        pltpu.VMEM((B,), jnp.int32),
        pltpu.VMEM((B,), jnp.int32),
        pltpu.VMEM((B,), jnp.int32),
        pltpu.SemaphoreType.DMA,
        pltpu.VMEM((B, 16), jnp.float32),
    ],
)


def kernel(x, edge_index, W, a_src, a_trg, W_skip, bias):
    f32 = jnp.float32

    eye_mask = jnp.kron(jnp.eye(H, dtype=f32), jnp.ones((F, 1), f32))
    a_src_m = eye_mask * a_src.reshape(HF, 1)
    a_trg_m = eye_mask * a_trg.reshape(HF, 1)
    ab = jnp.concatenate([a_src_m, a_trg_m], axis=1)
    expand = jnp.concatenate(
        [jnp.kron(jnp.eye(H, dtype=f32), jnp.ones((1, F), f32)),
         jnp.zeros((8, HF), f32)], axis=0)

    R = 1000
    grid = N // R
    proj, sboth, skip = pl.pallas_call(
        _pre_body,
        grid=(grid,),
        in_specs=[
            pl.BlockSpec((R, D_IN), lambda i: (i, 0)),
            pl.BlockSpec((D_IN, HF), lambda i: (0, 0)),
            pl.BlockSpec((D_IN, HF), lambda i: (0, 0)),
            pl.BlockSpec((HF, 16), lambda i: (0, 0)),
        ],
        out_specs=[
            pl.BlockSpec((R, HF), lambda i: (i, 0)),
            pl.BlockSpec((R, 16), lambda i: (i, 0)),
            pl.BlockSpec((R, HF), lambda i: (i, 0)),
        ],
        out_shape=[
            jax.ShapeDtypeStruct((N, HF), f32),
            jax.ShapeDtypeStruct((N, 16), f32),
            jax.ShapeDtypeStruct((N, HF), f32),
        ],
    )(x, W, W_skip, ab)

    src = edge_index[0]
    trg = edge_index[1]
    pad = E_ALLOC - E
    src_p = jnp.concatenate([src, jnp.zeros((pad,), jnp.int32)])
    trg_g = jnp.concatenate([trg, jnp.zeros((pad,), jnp.int32)])
    trg_s = jnp.concatenate([trg, jnp.full((pad,), N, jnp.int32)])

    num, den = _sc_edge(src_p, trg_g, trg_s, sboth, proj)

    out = pl.pallas_call(
        _post_body,
        grid=(grid,),
        in_specs=[
            pl.BlockSpec((R, HF), lambda i: (i, 0)),
            pl.BlockSpec((R, HF), lambda i: (i, 0)),
            pl.BlockSpec((R, 16), lambda i: (i, 0)),
            pl.BlockSpec((R, 16), lambda i: (i, 0)),
            pl.BlockSpec((R, HF), lambda i: (i, 0)),
            pl.BlockSpec((1, HF), lambda i: (0, 0)),
            pl.BlockSpec((16, HF), lambda i: (0, 0)),
        ],
        out_specs=pl.BlockSpec((R, HF), lambda i: (i, 0)),
        out_shape=jax.ShapeDtypeStruct((N, HF), f32),
    )(num[0, :N], num[1, :N], den[0, :N], den[1, :N], skip,
      bias.reshape(1, HF), expand)
    return out


# speedup vs baseline: 68.5163x; 1.2147x over previous
import functools

import jax
import jax.numpy as jnp
from jax import lax
from jax.experimental import pallas as pl
from jax.experimental.pallas import tpu as pltpu
from jax.experimental.pallas import tpu_sc as plsc

N = 10000
E = 320000
D_IN = 128
H = 8
F = 16
HF = H * F

NC = 2
NS = 16
NW = NC * NS
B = 64
NB = -(-E // (NW * B))
while NB % 3 != 2:
    NB += 1
E_PAD = NW * NB * B
E_ALLOC = E_PAD + B
N_PAD = 10240
_RPS = N_PAD // NS


def _pre_body(x_ref, w_ref, wskip_ref, ab_ref,
              proj_ref, sboth_ref, skip_ref):
    xb = x_ref[...]
    p = jnp.dot(xb, w_ref[...], preferred_element_type=jnp.float32)
    proj_ref[...] = p
    sboth_ref[...] = jnp.dot(p, ab_ref[...], preferred_element_type=jnp.float32)
    skip_ref[...] = jnp.dot(xb, wskip_ref[...], preferred_element_type=jnp.float32)


def _post_body(n0_ref, n1_ref, d0_ref, d1_ref, skip_ref, bias_ref, exp_ref,
               out_ref):
    den = d0_ref[...] + d1_ref[...]
    dinv = 1.0 / (den + 1e-16)
    dinv128 = jnp.dot(dinv, exp_ref[...], preferred_element_type=jnp.float32)
    v = (n0_ref[...] + n1_ref[...]) * dinv128 + skip_ref[...] + bias_ref[...]
    out_ref[...] = jnp.where(v > 0.0, v, jnp.exp(jnp.minimum(v, 0.0)) - 1.0)


def _sc_body(src_hbm, trgg_hbm, trgs_hbm, sboth_hbm, proj_hbm,
             num_hbm, den_hbm,
             sh_num, sh_den, zbuf, zden,
             prow0, ssrc0, strg0, exps0, srcv0, trgg0, trgs0, gsem0, ssem0,
             prow1, ssrc1, strg1, exps1, srcv1, trgg1, trgs1, gsem1, ssem1,
             prow2, ssrc2, strg2, exps2, srcv2, trgg2, trgs2, gsem2, ssem2):
    c = lax.axis_index("c")
    s = lax.axis_index("s")

    zeros = jnp.zeros((16,), jnp.float32)

    @pl.loop(0, 16)
    def _zero_rows(r):
        for j in range(8):
            zbuf[r, pl.ds(j * 16, 16)] = zeros
        zden[r, :] = zeros

    r0 = s * 320
    r1 = (NS + s) * 320

    @pl.loop(0, 20)
    def _zero_copy(t):
        pltpu.sync_copy(zbuf, sh_num.at[pl.ds(r0 + t * 16, 16)])
        pltpu.sync_copy(zden, sh_den.at[pl.ds(r0 + t * 16, 16)])
        pltpu.sync_copy(zbuf, sh_num.at[pl.ds(r1 + t * 16, 16)])
        pltpu.sync_copy(zden, sh_den.at[pl.ds(r1 + t * 16, 16)])

    plsc.subcore_barrier()

    base = (c * NS + s) * (NB * B)
    bufs = ((prow0, ssrc0, strg0, exps0, srcv0, trgg0, trgs0, gsem0, ssem0),
            (prow1, ssrc1, strg1, exps1, srcv1, trgg1, trgs1, gsem1, ssem1),
            (prow2, ssrc2, strg2, exps2, srcv2, trgg2, trgs2, gsem2, ssem2))

    def _fetch(blk, bset):
        prow, ssrc, strg, exps, srcv, trgg, trgs, gsem, ssem = bset
        off = base + blk * B
        pltpu.sync_copy(src_hbm.at[pl.ds(off, B)], srcv)
        pltpu.sync_copy(trgg_hbm.at[pl.ds(off, B)], trgg)
        pltpu.sync_copy(trgs_hbm.at[pl.ds(off, B)], trgs)
        pltpu.async_copy(sboth_hbm.at[srcv], ssrc, gsem)
        pltpu.async_copy(sboth_hbm.at[trgg], strg, gsem)
        pltpu.async_copy(proj_hbm.at[srcv], prow, gsem)

    def _drain_gather(bset):
        prow, ssrc, strg, exps, srcv, trgg, trgs, gsem, ssem = bset
        pltpu.make_async_copy(sboth_hbm.at[srcv], ssrc, gsem).wait()
        pltpu.make_async_copy(sboth_hbm.at[trgg], strg, gsem).wait()
        pltpu.make_async_copy(proj_hbm.at[srcv], prow, gsem).wait()

    def _scatter(bset):
        prow, ssrc, strg, exps, srcv, trgg, trgs, gsem, ssem = bset
        pltpu.async_copy(exps, sh_den.at[trgs], ssem, add=True)
        pltpu.async_copy(prow, sh_num.at[trgs], ssem, add=True)

    def _drain_scatter(bset):
        prow, ssrc, strg, exps, srcv, trgg, trgs, gsem, ssem = bset
        pltpu.make_async_copy(exps, sh_den.at[trgs], ssem).wait()
        pltpu.make_async_copy(prow, sh_num.at[trgs], ssem).wait()

    perm = jnp.minimum(lax.iota(jnp.int32, 16) + 8, 15)
    dnums = lax.GatherDimensionNumbers(
        offset_dims=(), collapsed_slice_dims=(0,), start_index_map=(0,))

    def _compute(bset):
        prow, ssrc, strg, exps, srcv, trgg, trgs, gsem, ssem = bset

        @pl.loop(0, B, unroll=2)
        def _edge(e):
            a = ssrc[e, :]
            b = strg[e, :]
            bp = lax.gather(b, perm[:, None], dnums, (1,),
                            mode=lax.GatherScatterMode.PROMISE_IN_BOUNDS)
            z = a + bp
            zl = jnp.where(z > 0.0, z, 0.2 * z)
            ex = jnp.exp(zl)
            exps[e, :] = ex
            for h in range(H):
                sc = ex[h]
                prow[e, pl.ds(h * F, F)] = prow[e, pl.ds(h * F, F)] * sc

    _fetch(0, bufs[0])
    _fetch(1, bufs[1])
    _drain_gather(bufs[0])
    _compute(bufs[0])
    _scatter(bufs[0])
    _fetch(2, bufs[2])
    _drain_gather(bufs[1])
    _compute(bufs[1])
    _scatter(bufs[1])

    @pl.loop(2, NB, step=3)
    def _edge_block(blk):
        for j in range(3):
            cur = bufs[(2 + j) % 3]
            nxt = bufs[(2 + j + 1) % 3]
            _drain_scatter(nxt)
            _fetch(blk + j + 1, nxt)
            _drain_gather(cur)
            _compute(cur)
            _scatter(cur)

    _drain_scatter(bufs[(NB - 2) % 3])
    _drain_scatter(bufs[(NB - 1) % 3])
    _drain_gather(bufs[NB % 3])

    plsc.subcore_barrier()

    w0 = s * _RPS
    pltpu.sync_copy(sh_num.at[pl.ds(w0, _RPS)], num_hbm.at[c, pl.ds(w0, _RPS), :])
    pltpu.sync_copy(sh_den.at[pl.ds(w0, _RPS)], den_hbm.at[c, pl.ds(w0, _RPS), :])


_sc_edge = pl.kernel(
    _sc_body,
    out_type=[
        jax.ShapeDtypeStruct((NC, N_PAD, HF), jnp.float32),
        jax.ShapeDtypeStruct((NC, N_PAD, 16), jnp.float32),
    ],
    mesh=plsc.VectorSubcoreMesh(core_axis_name="c", subcore_axis_name="s"),
    compiler_params=pltpu.CompilerParams(use_tc_tiling_on_sc=False),
    scratch_types=[
        pltpu.VMEM_SHARED((N_PAD, HF), jnp.float32),
        pltpu.VMEM_SHARED((N_PAD, 16), jnp.float32),
        pltpu.VMEM((16, HF), jnp.float32),
        pltpu.VMEM((16, 16), jnp.float32),
    ] + [
        t
        for _ in range(3)
        for t in (
            pltpu.VMEM((B, HF), jnp.float32),
            pltpu.VMEM((B, 16), jnp.float32),
            pltpu.VMEM((B, 16), jnp.float32),
            pltpu.VMEM((B, 16), jnp.float32),
            pltpu.VMEM((B,), jnp.int32),
            pltpu.VMEM((B,), jnp.int32),
            pltpu.VMEM((B,), jnp.int32),
            pltpu.SemaphoreType.DMA,
            pltpu.SemaphoreType.DMA,
        )
    ],
)


def kernel(x, edge_index, W, a_src, a_trg, W_skip, bias):
    f32 = jnp.float32

    eye_mask = jnp.kron(jnp.eye(H, dtype=f32), jnp.ones((F, 1), f32))
    a_src_m = eye_mask * a_src.reshape(HF, 1)
    a_trg_m = eye_mask * a_trg.reshape(HF, 1)
    ab = jnp.concatenate([a_src_m, a_trg_m], axis=1)
    expand = jnp.concatenate(
        [jnp.kron(jnp.eye(H, dtype=f32), jnp.ones((1, F), f32)),
         jnp.zeros((8, HF), f32)], axis=0)

    R = 1000
    grid = N // R
    proj, sboth, skip = pl.pallas_call(
        _pre_body,
        grid=(grid,),
        in_specs=[
            pl.BlockSpec((R, D_IN), lambda i: (i, 0)),
            pl.BlockSpec((D_IN, HF), lambda i: (0, 0)),
            pl.BlockSpec((D_IN, HF), lambda i: (0, 0)),
            pl.BlockSpec((HF, 16), lambda i: (0, 0)),
        ],
        out_specs=[
            pl.BlockSpec((R, HF), lambda i: (i, 0)),
            pl.BlockSpec((R, 16), lambda i: (i, 0)),
            pl.BlockSpec((R, HF), lambda i: (i, 0)),
        ],
        out_shape=[
            jax.ShapeDtypeStruct((N, HF), f32),
            jax.ShapeDtypeStruct((N, 16), f32),
            jax.ShapeDtypeStruct((N, HF), f32),
        ],
    )(x, W, W_skip, ab)

    src = edge_index[0]
    trg = edge_index[1]
    pad = E_ALLOC - E
    src_p = jnp.concatenate([src, jnp.zeros((pad,), jnp.int32)])
    trg_g = jnp.concatenate([trg, jnp.zeros((pad,), jnp.int32)])
    trg_s = jnp.concatenate([trg, jnp.full((pad,), N, jnp.int32)])

    num, den = _sc_edge(src_p, trg_g, trg_s, sboth, proj)

    out = pl.pallas_call(
        _post_body,
        grid=(grid,),
        in_specs=[
            pl.BlockSpec((R, HF), lambda i: (i, 0)),
            pl.BlockSpec((R, HF), lambda i: (i, 0)),
            pl.BlockSpec((R, 16), lambda i: (i, 0)),
            pl.BlockSpec((R, 16), lambda i: (i, 0)),
            pl.BlockSpec((R, HF), lambda i: (i, 0)),
            pl.BlockSpec((1, HF), lambda i: (0, 0)),
            pl.BlockSpec((16, HF), lambda i: (0, 0)),
        ],
        out_specs=pl.BlockSpec((R, HF), lambda i: (i, 0)),
        out_shape=jax.ShapeDtypeStruct((N, HF), f32),
    )(num[0, :N], num[1, :N], den[0, :N], den[1, :N], skip,
      bias.reshape(1, HF), expand)
    return out


# speedup vs baseline: 92.4235x; 1.3489x over previous
import jax
import jax.numpy as jnp
from jax import lax
from jax.experimental import pallas as pl
from jax.experimental.pallas import tpu as pltpu
from jax.experimental.pallas import tpu_sc as plsc

N = 10000
E = 320000
D_IN = 128
H = 8
F = 16
HF = H * F

NC = 2
NS = 16
NW = NC * NS
B = 56
CH = 6
NB = -(-E // (NW * B))
while NB % (2 * CH) != 0:
    NB += 1
NCH = NB // CH
E_PAD = NW * NB * B
E_ALLOC = (NW * NB + CH) * B
N_PAD = 10240
_RPS = N_PAD // NS
SB_ROWS = N + 8


def _pre_body(x_ref, w_ref, wskip_ref, ab_ref,
              proj_ref, sboth_ref, skip_ref):
    xb = x_ref[...]
    p = jnp.dot(xb, w_ref[...], preferred_element_type=jnp.float32)
    proj_ref[...] = p
    sboth_ref[...] = jnp.dot(p, ab_ref[...], preferred_element_type=jnp.float32)
    skip_ref[...] = jnp.dot(xb, wskip_ref[...], preferred_element_type=jnp.float32)


def _post_body(n0_ref, n1_ref, d0_ref, d1_ref, skip_ref, bias_ref, exp_ref,
               out_ref):
    den = d0_ref[...] + d1_ref[...]
    dinv = 1.0 / (den + 1e-16)
    dinv128 = jnp.dot(dinv, exp_ref[...], preferred_element_type=jnp.float32)
    v = (n0_ref[...] + n1_ref[...]) * dinv128 + skip_ref[...] + bias_ref[...]
    out_ref[...] = jnp.where(v > 0.0, v, jnp.exp(jnp.minimum(v, 0.0)) - 1.0)


def _sc_body(src2_hbm, trg2_hbm, sboth_hbm, proj_hbm,
             num_hbm, den_hbm,
             sh_num, sh_den, zbuf, zden,
             prow0, ssrc0, strg0, exps0, gsem0, ssem0,
             prow1, ssrc1, strg1, exps1, gsem1, ssem1,
             prow2, ssrc2, strg2, exps2, gsem2, ssem2,
             isrc0, itrg0, csem0, isrc1, itrg1, csem1):
    c = lax.axis_index("c")
    s = lax.axis_index("s")

    zeros = jnp.zeros((16,), jnp.float32)

    @pl.loop(0, 16)
    def _zero_rows(r):
        for j in range(8):
            zbuf[r, pl.ds(j * 16, 16)] = zeros
        zden[r, :] = zeros

    r0 = s * 320
    r1 = (NS + s) * 320

    @pl.loop(0, 20)
    def _zero_copy(t):
        pltpu.sync_copy(zbuf, sh_num.at[pl.ds(r0 + t * 16, 16)])
        pltpu.sync_copy(zden, sh_den.at[pl.ds(r0 + t * 16, 16)])
        pltpu.sync_copy(zbuf, sh_num.at[pl.ds(r1 + t * 16, 16)])
        pltpu.sync_copy(zden, sh_den.at[pl.ds(r1 + t * 16, 16)])

    plsc.subcore_barrier()

    wrow = (c * NS + s) * NB
    bufs = ((prow0, ssrc0, strg0, exps0, gsem0, ssem0),
            (prow1, ssrc1, strg1, exps1, gsem1, ssem1),
            (prow2, ssrc2, strg2, exps2, gsem2, ssem2))
    ibufs = ((isrc0, itrg0, csem0), (isrc1, itrg1, csem1))

    def _chunk_fetch(ck, p):
        isrc, itrg, csem = ibufs[p]
        pltpu.async_copy(src2_hbm.at[pl.ds(wrow + ck * CH, CH)], isrc, csem)
        pltpu.async_copy(trg2_hbm.at[pl.ds(wrow + ck * CH, CH)], itrg, csem)

    def _chunk_drain(p):
        isrc, itrg, csem = ibufs[p]
        pltpu.make_async_copy(src2_hbm.at[pl.ds(wrow, CH)], isrc, csem).wait()
        pltpu.make_async_copy(trg2_hbm.at[pl.ds(wrow, CH)], itrg, csem).wait()

    def _fetch(bset, p, j):
        prow, ssrc, strg, exps, gsem, ssem = bset
        isrc, itrg, _ = ibufs[p]
        pltpu.async_copy(sboth_hbm.at[isrc.at[j]], ssrc, gsem)
        pltpu.async_copy(sboth_hbm.at[itrg.at[j]], strg, gsem)
        pltpu.async_copy(proj_hbm.at[isrc.at[j]], prow, gsem)

    def _drain_gather(bset):
        prow, ssrc, strg, exps, gsem, ssem = bset
        idx = ibufs[0][0].at[0]
        pltpu.make_async_copy(sboth_hbm.at[idx], ssrc, gsem).wait()
        pltpu.make_async_copy(sboth_hbm.at[idx], strg, gsem).wait()
        pltpu.make_async_copy(proj_hbm.at[idx], prow, gsem).wait()

    def _scatter(bset, p, j):
        prow, ssrc, strg, exps, gsem, ssem = bset
        itrg = ibufs[p][1]
        pltpu.async_copy(exps, sh_den.at[itrg.at[j]], ssem, add=True)
        pltpu.async_copy(prow, sh_num.at[itrg.at[j]], ssem, add=True)

    def _drain_scatter(bset):
        prow, ssrc, strg, exps, gsem, ssem = bset
        idx = ibufs[0][1].at[0]
        pltpu.make_async_copy(exps, sh_den.at[idx], ssem).wait()
        pltpu.make_async_copy(prow, sh_num.at[idx], ssem).wait()

    perm = jnp.minimum(lax.iota(jnp.int32, 16) + 8, 15)
    dnums = lax.GatherDimensionNumbers(
        offset_dims=(), collapsed_slice_dims=(0,), start_index_map=(0,))

    def _compute(bset):
        prow, ssrc, strg, exps, gsem, ssem = bset

        @pl.loop(0, B, unroll=2)
        def _edge(e):
            a = ssrc[e, :]
            b = strg[e, :]
            bp = lax.gather(b, perm[:, None], dnums, (1,),
                            mode=lax.GatherScatterMode.PROMISE_IN_BOUNDS)
            z = a + bp
            zl = jnp.where(z > 0.0, z, 0.2 * z)
            ex = jnp.exp(zl)
            exps[e, :] = ex
            for h in range(H):
                sc = ex[h]
                prow[e, pl.ds(h * F, F)] = prow[e, pl.ds(h * F, F)] * sc

    def _chunk_body(ck, p, drain_early):
        for j in range(CH):
            X = bufs[j % 3]
            Y = bufs[(j + 1) % 3]
            if drain_early or j >= 2:
                _drain_scatter(Y)
            if j == CH - 1:
                _chunk_drain(1 - p)
                _fetch(Y, 1 - p, 0)
            else:
                _fetch(Y, p, j + 1)
            _drain_gather(X)
            _compute(X)
            _scatter(X, p, j)
            if j == 1:
                _chunk_fetch(ck + 1, 1 - p)

    _chunk_fetch(0, 0)
    _chunk_drain(0)
    _fetch(bufs[0], 0, 0)
    _chunk_body(0, 0, drain_early=False)
    _chunk_body(1, 1, drain_early=True)

    @pl.loop(2, NCH, step=2)
    def _chunks(cc):
        _chunk_body(cc, 0, drain_early=True)
        _chunk_body(cc + 1, 1, drain_early=True)

    _drain_scatter(bufs[(NB - 2) % 3])
    _drain_scatter(bufs[(NB - 1) % 3])
    _drain_gather(bufs[NB % 3])

    plsc.subcore_barrier()

    w0 = s * _RPS
    pltpu.sync_copy(sh_num.at[pl.ds(w0, _RPS)], num_hbm.at[c, pl.ds(w0, _RPS), :])
    pltpu.sync_copy(sh_den.at[pl.ds(w0, _RPS)], den_hbm.at[c, pl.ds(w0, _RPS), :])


_sc_edge = pl.kernel(
    _sc_body,
    out_type=[
        jax.ShapeDtypeStruct((NC, N_PAD, HF), jnp.float32),
        jax.ShapeDtypeStruct((NC, N_PAD, 16), jnp.float32),
    ],
    mesh=plsc.VectorSubcoreMesh(core_axis_name="c", subcore_axis_name="s"),
    compiler_params=pltpu.CompilerParams(use_tc_tiling_on_sc=False),
    scratch_types=[
        pltpu.VMEM_SHARED((N_PAD, HF), jnp.float32),
        pltpu.VMEM_SHARED((N_PAD, 16), jnp.float32),
        pltpu.VMEM((16, HF), jnp.float32),
        pltpu.VMEM((16, 16), jnp.float32),
    ] + [
        t
        for _ in range(3)
        for t in (
            pltpu.VMEM((B, HF), jnp.float32),
            pltpu.VMEM((B, 16), jnp.float32),
            pltpu.VMEM((B, 16), jnp.float32),
            pltpu.VMEM((B, 16), jnp.float32),
            pltpu.SemaphoreType.DMA,
            pltpu.SemaphoreType.DMA,
        )
    ] + [
        t
        for _ in range(2)
        for t in (
            pltpu.VMEM((CH, B), jnp.int32),
            pltpu.VMEM((CH, B), jnp.int32),
            pltpu.SemaphoreType.DMA,
        )
    ],
)


def kernel(x, edge_index, W, a_src, a_trg, W_skip, bias):
    f32 = jnp.float32

    eye_mask = jnp.kron(jnp.eye(H, dtype=f32), jnp.ones((F, 1), f32))
    a_src_m = eye_mask * a_src.reshape(HF, 1)
    a_trg_m = eye_mask * a_trg.reshape(HF, 1)
    ab = jnp.concatenate([a_src_m, a_trg_m], axis=1)
    expand = jnp.concatenate(
        [jnp.kron(jnp.eye(H, dtype=f32), jnp.ones((1, F), f32)),
         jnp.zeros((8, HF), f32)], axis=0)

    R = 1000
    grid = N // R
    proj, sboth, skip = pl.pallas_call(
        _pre_body,
        grid=(grid,),
        in_specs=[
            pl.BlockSpec((R, D_IN), lambda i: (i, 0)),
            pl.BlockSpec((D_IN, HF), lambda i: (0, 0)),
            pl.BlockSpec((D_IN, HF), lambda i: (0, 0)),
            pl.BlockSpec((HF, 16), lambda i: (0, 0)),
        ],
        out_specs=[
            pl.BlockSpec((R, HF), lambda i: (i, 0)),
            pl.BlockSpec((R, 16), lambda i: (i, 0)),
            pl.BlockSpec((R, HF), lambda i: (i, 0)),
        ],
        out_shape=[
            jax.ShapeDtypeStruct((N, HF), f32),
            jax.ShapeDtypeStruct((N, 16), f32),
            jax.ShapeDtypeStruct((N, HF), f32),
        ],
    )(x, W, W_skip, ab)

    sboth_p = jnp.pad(sboth, ((0, SB_ROWS - N), (0, 0)))

    src = edge_index[0]
    trg = edge_index[1]
    pad = E_ALLOC - E
    src2 = jnp.concatenate([src, jnp.zeros((pad,), jnp.int32)]).reshape(-1, B)
    trg2 = jnp.concatenate([trg, jnp.full((pad,), N, jnp.int32)]).reshape(-1, B)

    num, den = _sc_edge(src2, trg2, sboth_p, proj)

    out = pl.pallas_call(
        _post_body,
        grid=(grid,),
        in_specs=[
            pl.BlockSpec((R, HF), lambda i: (i, 0)),
            pl.BlockSpec((R, HF), lambda i: (i, 0)),
            pl.BlockSpec((R, 16), lambda i: (i, 0)),
            pl.BlockSpec((R, 16), lambda i: (i, 0)),
            pl.BlockSpec((R, HF), lambda i: (i, 0)),
            pl.BlockSpec((1, HF), lambda i: (0, 0)),
            pl.BlockSpec((16, HF), lambda i: (0, 0)),
        ],
        out_specs=pl.BlockSpec((R, HF), lambda i: (i, 0)),
        out_shape=jax.ShapeDtypeStruct((N, HF), f32),
    )(num[0, :N], num[1, :N], den[0, :N], den[1, :N], skip,
      bias.reshape(1, HF), expand)
    return out
